# baseline (speedup 1.0000x reference)
import functools

import jax
import jax.numpy as jnp
from jax import lax
from jax.experimental import pallas as pl
from jax.experimental.pallas import tpu as pltpu
from jax.experimental.pallas import tpu_sc as plsc

D = 32
B = 16384
NC, NS = 2, 16
NW = NC * NS
ROWS_W = B // NW
R = 16
NBUF = 2

_mesh = plsc.VectorSubcoreMesh(core_axis_name="c", subcore_axis_name="s")


def _make_lookup(K):
    @functools.partial(
        pl.kernel,
        out_type=jax.ShapeDtypeStruct((B, K, D), jnp.float32),
        mesh=_mesh,
        scratch_types=(
            [pltpu.VMEM((R * K,), jnp.int32) for _ in range(NBUF)]
            + [pltpu.VMEM((R * K, D), jnp.float32) for _ in range(NBUF)]
            + [pltpu.SemaphoreType.DMA((NBUF,)),
               pltpu.SemaphoreType.DMA((NBUF,)),
               pltpu.SemaphoreType.DMA((NBUF,))]
        ),
        compiler_params=pltpu.CompilerParams(use_tc_tiling_on_sc=False),
    )
    def _lookup(idx_hbm, table, out_hbm, *scratch):
        idx_bufs = scratch[:NBUF]
        row_bufs = scratch[NBUF:2 * NBUF]
        sem_idx, sem_g, sem_wb = scratch[2 * NBUF:]
        wid = lax.axis_index("s") * NC + lax.axis_index("c")
        row0 = wid * ROWS_W
        nch = ROWS_W // R
        chunk = R * K
        base_w = row0 * K

        def wb_start(b, rbase):
            for r in range(R):
                pltpu.async_copy(row_bufs[b].at[pl.ds(r * K, K), :],
                                 out_hbm.at[rbase + r], sem_wb.at[b])

        def wb_wait(b, rbase):
            for r in range(R):
                pltpu.make_async_copy(row_bufs[b].at[pl.ds(r * K, K), :],
                                      out_hbm.at[rbase + r],
                                      sem_wb.at[b]).wait()

        for b in range(NBUF):
            pltpu.async_copy(idx_hbm.at[pl.ds(base_w + b * chunk, chunk)],
                             idx_bufs[b], sem_idx.at[b])

        def body(p, carry):
            for b in range(NBUF):
                c = p * NBUF + b
                base = base_w + c * chunk
                rbase = row0 + c * R
                pltpu.make_async_copy(idx_hbm.at[pl.ds(base, chunk)],
                                      idx_bufs[b], sem_idx.at[b]).wait()

                @pl.when(p > 0)
                def _():
                    wb_wait(b, rbase)

                pltpu.async_copy(table.at[idx_bufs[b]], row_bufs[b],
                                 sem_g.at[b]).wait()
                wb_start(b, rbase)

                @pl.when(c + NBUF < nch)
                def _():
                    pltpu.async_copy(
                        idx_hbm.at[pl.ds(base + NBUF * chunk, chunk)],
                        idx_bufs[b], sem_idx.at[b])

            return carry

        lax.fori_loop(0, nch // NBUF, body, 0)
        for b in range(NBUF):
            wb_wait(b, row0)

    return _lookup


_lookup_in = _make_lookup(50)
_lookup_sup = _make_lookup(20)


def kernel(input, support, W):
    out_in = _lookup_in(input.reshape(-1), W)
    sup_flat, out_in = lax.optimization_barrier((support.reshape(-1), out_in))
    out_sup = _lookup_sup(sup_flat, W)
    return (out_in, out_sup)


# speedup vs baseline: 1.1502x; 1.1502x over previous
import functools

import jax
import jax.numpy as jnp
from jax import lax
from jax.experimental import pallas as pl
from jax.experimental.pallas import tpu as pltpu
from jax.experimental.pallas import tpu_sc as plsc

D = 32
B = 16384
NC, NS = 2, 16
NW = NC * NS
ROWS_W = B // NW
R = 16
NBUF = 2

_mesh = plsc.VectorSubcoreMesh(core_axis_name="c", subcore_axis_name="s")


def _make_lookup(K):
    @functools.partial(
        pl.kernel,
        out_type=jax.ShapeDtypeStruct((B, K, D), jnp.float32),
        mesh=_mesh,
        scratch_types=(
            [pltpu.VMEM((R * K,), jnp.int32) for _ in range(NBUF)]
            + [pltpu.VMEM((R * K, D), jnp.float32) for _ in range(NBUF)]
            + [pltpu.SemaphoreType.DMA((NBUF,)),
               pltpu.SemaphoreType.DMA((NBUF,)),
               pltpu.SemaphoreType.DMA((NBUF,))]
        ),
        compiler_params=pltpu.CompilerParams(use_tc_tiling_on_sc=False),
    )
    def _lookup(idx_hbm, table, out_hbm, *scratch):
        idx_bufs = scratch[:NBUF]
        row_bufs = scratch[NBUF:2 * NBUF]
        sem_idx, sem_g, sem_wb = scratch[2 * NBUF:]
        wid = lax.axis_index("s") * NC + lax.axis_index("c")
        row0 = wid * ROWS_W
        nch = ROWS_W // R
        chunk = R * K
        base_w = row0 * K

        def wb_start(b, rbase):
            for r in range(R):
                pltpu.async_copy(row_bufs[b].at[pl.ds(r * K, K), :],
                                 out_hbm.at[rbase + r], sem_wb.at[b])

        def wb_wait(b, rbase):
            for r in range(R):
                pltpu.make_async_copy(row_bufs[b].at[pl.ds(r * K, K), :],
                                      out_hbm.at[rbase + r],
                                      sem_wb.at[b]).wait()

        for b in range(NBUF):
            pltpu.async_copy(idx_hbm.at[pl.ds(base_w + b * chunk, chunk)],
                             idx_bufs[b], sem_idx.at[b])

        def body(p, carry):
            for b in range(NBUF):
                c = p * NBUF + b
                base = base_w + c * chunk
                rbase = row0 + c * R
                pltpu.make_async_copy(idx_hbm.at[pl.ds(base, chunk)],
                                      idx_bufs[b], sem_idx.at[b]).wait()

                @pl.when(p > 0)
                def _():
                    wb_wait(b, rbase)

                pltpu.async_copy(table.at[idx_bufs[b]], row_bufs[b],
                                 sem_g.at[b]).wait()
                wb_start(b, rbase)

                @pl.when(c + NBUF < nch)
                def _():
                    pltpu.async_copy(
                        idx_hbm.at[pl.ds(base + NBUF * chunk, chunk)],
                        idx_bufs[b], sem_idx.at[b])

            return carry

        lax.fori_loop(0, nch // NBUF, body, 0)
        for b in range(NBUF):
            wb_wait(b, row0)

    return _lookup


_lookup_in = _make_lookup(50)
_lookup_sup = _make_lookup(20)


def kernel(input, support, W):
    out_in = _lookup_in(input.reshape(-1), W)
    out_sup = _lookup_sup(support.reshape(-1), W)
    return (out_in, out_sup)
